# baseline (speedup 1.0000x reference)
import functools

import jax
import jax.numpy as jnp
from jax.experimental import pallas as pl
from jax.experimental.pallas import tpu as pltpu

_BN_EPS = 1e-5
_LEAD = 128
_GAP = 192
_BATCH = 2


def _fused_kernel(x_ref, w_ref, g_ref, b_ref, out_ref, xs_ref, ys_ref,
                  st_ref, mask_ref, *, h, w, n):
    n_pix = h * w
    halo = w + 1
    wide = 2 * n_pix + _GAP
    img1 = _LEAD + n_pix + _GAP
    phase = pl.program_id(0)
    step = pl.program_id(1)

    @pl.when((phase == 0) & (step == 0))
    def _init_constants():
        cin = x_ref.shape[1]
        xs_ref[:, :_LEAD] = jnp.zeros((cin, _LEAD), xs_ref.dtype)
        xs_ref[:, _LEAD + n_pix:img1] = jnp.zeros(
            (cin, _GAP), xs_ref.dtype
        )
        xs_ref[:, img1 + n_pix:] = jnp.zeros(
            (cin, xs_ref.shape[1] - img1 - n_pix), xs_ref.dtype
        )
        c = jax.lax.broadcasted_iota(jnp.int32, (8, wide), 1)
        col = jnp.where(c < n_pix, c, c - n_pix - _GAP) % w
        valid = (c < n_pix) | (c >= n_pix + _GAP)
        mask_ref[0:8] = (valid & (col > 0)).astype(jnp.float32)
        mask_ref[8:16] = (valid & (col < w - 1)).astype(jnp.float32)

    @pl.when(phase == 0)
    def _conv_phase():
        xs_ref[:, _LEAD:_LEAD + n_pix] = x_ref[0].astype(xs_ref.dtype)
        xs_ref[:, img1:img1 + n_pix] = x_ref[1].astype(xs_ref.dtype)
        xs = xs_ref[...]
        m0 = mask_ref[0:1]
        m2 = mask_ref[8:9]

        accs = []
        for kw in range(3):
            acc = jnp.zeros((ys_ref.shape[1], wide), jnp.float32)
            for kh in range(3):
                off = _LEAD - halo + kh * w + kw
                acc = acc + jnp.dot(
                    w_ref[3 * kh + kw], xs[:, off:off + wide],
                    preferred_element_type=jnp.float32,
                )
            accs.append(acc)
        acc = accs[1] + m0 * accs[0] + m2 * accs[2]

        a0 = acc[:, :n_pix]
        a1 = acc[:, n_pix + _GAP:]
        ys_ref[step * _BATCH] = a0.astype(ys_ref.dtype)
        ys_ref[step * _BATCH + 1] = a1.astype(ys_ref.dtype)
        ssum = (jnp.sum(a0, axis=1, keepdims=True)
                + jnp.sum(a1, axis=1, keepdims=True))
        ssq = (jnp.sum(a0 * a0, axis=1, keepdims=True)
               + jnp.sum(a1 * a1, axis=1, keepdims=True))
        part = jnp.concatenate([ssum, ssq], axis=1)

        @pl.when(step == 0)
        def _init():
            st_ref[...] = part

        @pl.when(step > 0)
        def _accum():
            st_ref[...] = st_ref[...] + part

    @pl.when(phase == 1)
    def _bn_phase():
        st = st_ref[...]
        mean = st[:, 0] / n
        var = jnp.maximum(st[:, 1] / n - mean * mean, 0.0)
        scale = g_ref[...][:, 0] * jax.lax.rsqrt(var + _BN_EPS)
        shift = b_ref[...][:, 0] - mean * scale
        for j in range(_BATCH):
            y = ys_ref[step * _BATCH + j].astype(jnp.float32)
            out_ref[j] = jnp.maximum(
                y * scale.reshape(-1, 1) + shift.reshape(-1, 1), 0.0
            )


def kernel(x_nchw, w_oihw, bias, gamma, beta):
    del bias
    N, C, H, W = x_nchw.shape
    Cout, _, KH, KW = w_oihw.shape
    assert KH == 3 and KW == 3

    P = H * W
    NS = N // _BATCH
    width = _LEAD + P + _GAP + P + _LEAD

    x_flat = x_nchw.reshape(N, C, P)
    wt = jnp.transpose(w_oihw, (2, 3, 0, 1)).reshape(9, Cout, C)
    wt = wt.astype(jnp.bfloat16)

    cparams = pltpu.CompilerParams(
        dimension_semantics=("arbitrary", "arbitrary"),
        vmem_limit_bytes=100 * 1024 * 1024,
    )

    out_flat = pl.pallas_call(
        functools.partial(_fused_kernel, h=H, w=W, n=N * P),
        out_shape=jax.ShapeDtypeStruct((N, Cout, P), jnp.float32),
        grid=(2, NS),
        in_specs=[
            pl.BlockSpec((_BATCH, C, P),
                         lambda p, i: (i * (1 - p) + (NS - 1) * p, 0, 0)),
            pl.BlockSpec((9, Cout, C), lambda p, i: (0, 0, 0)),
            pl.BlockSpec((Cout, 1), lambda p, i: (0, 0)),
            pl.BlockSpec((Cout, 1), lambda p, i: (0, 0)),
        ],
        out_specs=pl.BlockSpec((_BATCH, Cout, P), lambda p, i: (i * p, 0, 0)),
        scratch_shapes=[
            pltpu.VMEM((C, width), jnp.bfloat16),
            pltpu.VMEM((N, Cout, P), jnp.bfloat16),
            pltpu.VMEM((Cout, 2), jnp.float32),
            pltpu.VMEM((16, 2 * P + _GAP), jnp.float32),
        ],
        compiler_params=cparams,
    )(x_flat, wt, gamma.astype(jnp.float32).reshape(Cout, 1),
      beta.astype(jnp.float32).reshape(Cout, 1))
    return out_flat.reshape(N, Cout, H, W)


# speedup vs baseline: 1.0069x; 1.0069x over previous
import functools

import jax
import jax.numpy as jnp
from jax.experimental import pallas as pl
from jax.experimental.pallas import tpu as pltpu

_BN_EPS = 1e-5
_ALIGN = 128
_BATCH = 4


def _fused_kernel(x_ref, w_ref, g_ref, b_ref, out_ref, xs_ref, ys_ref,
                  st_ref, mask_ref, *, h, w, n):
    n_pix = h * w
    halo = w + 1
    phase = pl.program_id(0)
    step = pl.program_id(1)

    @pl.when((phase == 0) & (step == 0))
    def _init_constants():
        cin = x_ref.shape[1]
        xs_ref[:, :_ALIGN] = jnp.zeros((cin, _ALIGN), xs_ref.dtype)
        xs_ref[:, _ALIGN + n_pix:] = jnp.zeros(
            (cin, xs_ref.shape[1] - _ALIGN - n_pix), xs_ref.dtype
        )
        col = jax.lax.broadcasted_iota(jnp.int32, (8, n_pix), 1) % w
        mask_ref[0:8] = (col > 0).astype(jnp.float32)
        mask_ref[8:16] = (col < w - 1).astype(jnp.float32)

    @pl.when(phase == 0)
    def _conv_phase():
        m0 = mask_ref[0:1]
        m2 = mask_ref[8:9]
        part = jnp.zeros((st_ref.shape[0], 2), jnp.float32)
        for j in range(_BATCH):
            xs_ref[:, _ALIGN:_ALIGN + n_pix] = x_ref[j].astype(xs_ref.dtype)
            xs = xs_ref[...]
            accs = []
            for kw in range(3):
                acc = jnp.zeros((ys_ref.shape[1], n_pix), jnp.float32)
                for kh in range(3):
                    off = _ALIGN - halo + kh * w + kw
                    acc = acc + jnp.dot(
                        w_ref[3 * kh + kw], xs[:, off:off + n_pix],
                        preferred_element_type=jnp.float32,
                    )
                accs.append(acc)
            acc = accs[1] + m0 * accs[0] + m2 * accs[2]
            ys_ref[step * _BATCH + j] = acc.astype(ys_ref.dtype)
            ssum = jnp.sum(acc, axis=1, keepdims=True)
            ssq = jnp.sum(acc * acc, axis=1, keepdims=True)
            part = part + jnp.concatenate([ssum, ssq], axis=1)

        @pl.when(step == 0)
        def _init():
            st_ref[...] = part

        @pl.when(step > 0)
        def _accum():
            st_ref[...] = st_ref[...] + part

    @pl.when(phase == 1)
    def _bn_phase():
        st = st_ref[...]
        mean = st[:, 0] / n
        var = jnp.maximum(st[:, 1] / n - mean * mean, 0.0)
        scale = g_ref[...][:, 0] * jax.lax.rsqrt(var + _BN_EPS)
        shift = b_ref[...][:, 0] - mean * scale
        for j in range(_BATCH):
            y = ys_ref[step * _BATCH + j].astype(jnp.float32)
            out_ref[j] = jnp.maximum(
                y * scale.reshape(-1, 1) + shift.reshape(-1, 1), 0.0
            )


def kernel(x_nchw, w_oihw, bias, gamma, beta):
    del bias
    N, C, H, W = x_nchw.shape
    Cout, _, KH, KW = w_oihw.shape
    assert KH == 3 and KW == 3

    P = H * W
    NS = N // _BATCH

    x_flat = x_nchw.reshape(N, C, P)
    wt = jnp.transpose(w_oihw, (2, 3, 0, 1)).reshape(9, Cout, C)
    wt = wt.astype(jnp.bfloat16)

    cparams = pltpu.CompilerParams(
        dimension_semantics=("arbitrary", "arbitrary"),
        vmem_limit_bytes=100 * 1024 * 1024,
    )

    out_flat = pl.pallas_call(
        functools.partial(_fused_kernel, h=H, w=W, n=N * P),
        out_shape=jax.ShapeDtypeStruct((N, Cout, P), jnp.float32),
        grid=(2, NS),
        in_specs=[
            pl.BlockSpec((_BATCH, C, P),
                         lambda p, i: (i * (1 - p) + (NS - 1) * p, 0, 0)),
            pl.BlockSpec((9, Cout, C), lambda p, i: (0, 0, 0)),
            pl.BlockSpec((Cout, 1), lambda p, i: (0, 0)),
            pl.BlockSpec((Cout, 1), lambda p, i: (0, 0)),
        ],
        out_specs=pl.BlockSpec((_BATCH, Cout, P), lambda p, i: (i * p, 0, 0)),
        scratch_shapes=[
            pltpu.VMEM((C, _ALIGN + P + _ALIGN), jnp.bfloat16),
            pltpu.VMEM((N, Cout, P), jnp.bfloat16),
            pltpu.VMEM((Cout, 2), jnp.float32),
            pltpu.VMEM((16, P), jnp.float32),
        ],
        compiler_params=cparams,
    )(x_flat, wt, gamma.astype(jnp.float32).reshape(Cout, 1),
      beta.astype(jnp.float32).reshape(Cout, 1))
    return out_flat.reshape(N, Cout, H, W)


# speedup vs baseline: 1.0114x; 1.0045x over previous
import functools

import jax
import jax.numpy as jnp
from jax.experimental import pallas as pl
from jax.experimental.pallas import tpu as pltpu

_BN_EPS = 1e-5
_ALIGN = 128
_BATCH = 4


def _fused_kernel(x_ref, w_ref, g_ref, b_ref, out_ref, xs_ref, ys_ref,
                  st_ref, mask_ref, *, h, w, n, batch):
    n_pix = h * w
    halo = w + 1
    phase = pl.program_id(0)
    step = pl.program_id(1)

    @pl.when((phase == 0) & (step == 0))
    def _init_constants():
        cin = x_ref.shape[1]
        xs_ref[:, :_ALIGN] = jnp.zeros((cin, _ALIGN), xs_ref.dtype)
        xs_ref[:, _ALIGN + n_pix:] = jnp.zeros(
            (cin, xs_ref.shape[1] - _ALIGN - n_pix), xs_ref.dtype
        )
        col = jax.lax.broadcasted_iota(jnp.int32, (8, n_pix), 1) % w
        mask_ref[0:8] = (col > 0).astype(jnp.float32)
        mask_ref[8:16] = (col < w - 1).astype(jnp.float32)

    @pl.when(phase == 0)
    def _conv_phase():
        m0 = mask_ref[0:1]
        m2 = mask_ref[8:9]
        part = jnp.zeros((st_ref.shape[0], 2), jnp.float32)
        for j in range(batch):
            xs_ref[:, _ALIGN:_ALIGN + n_pix] = x_ref[j].astype(xs_ref.dtype)
            xs = xs_ref[...]
            accs = []
            for kw in range(3):
                acc = jnp.zeros((ys_ref.shape[1], n_pix), jnp.float32)
                for kh in range(3):
                    off = _ALIGN - halo + kh * w + kw
                    acc = acc + jnp.dot(
                        w_ref[3 * kh + kw], xs[:, off:off + n_pix],
                        preferred_element_type=jnp.float32,
                    )
                accs.append(acc)
            acc = accs[1] + m0 * accs[0] + m2 * accs[2]
            ys_ref[step * batch + j] = acc.astype(ys_ref.dtype)
            ssum = jnp.sum(acc, axis=1, keepdims=True)
            ssq = jnp.sum(acc * acc, axis=1, keepdims=True)
            part = part + jnp.concatenate([ssum, ssq], axis=1)

        @pl.when(step == 0)
        def _init():
            st_ref[...] = part

        @pl.when(step > 0)
        def _accum():
            st_ref[...] = st_ref[...] + part

    @pl.when(phase == 1)
    def _bn_phase():
        st = st_ref[...]
        mean = st[:, 0] / n
        var = jnp.maximum(st[:, 1] / n - mean * mean, 0.0)
        scale = g_ref[...][:, 0] * jax.lax.rsqrt(var + _BN_EPS)
        shift = b_ref[...][:, 0] - mean * scale
        for j in range(batch):
            y = ys_ref[step * batch + j].astype(jnp.float32)
            out_ref[j] = jnp.maximum(
                y * scale.reshape(-1, 1) + shift.reshape(-1, 1), 0.0
            )


def kernel(x_nchw, w_oihw, bias, gamma, beta):
    del bias
    N, C, H, W = x_nchw.shape
    Cout, _, KH, KW = w_oihw.shape
    assert KH == 3 and KW == 3

    P = H * W
    batch = _BATCH if N % _BATCH == 0 else 1
    NS = N // batch

    x_flat = x_nchw.reshape(N, C, P)
    wt = jnp.transpose(w_oihw, (2, 3, 0, 1)).reshape(9, Cout, C)
    wt = wt.astype(jnp.bfloat16)

    cparams = pltpu.CompilerParams(
        dimension_semantics=("arbitrary", "arbitrary"),
        vmem_limit_bytes=100 * 1024 * 1024,
    )

    out_flat = pl.pallas_call(
        functools.partial(_fused_kernel, h=H, w=W, n=N * P, batch=batch),
        out_shape=jax.ShapeDtypeStruct((N, Cout, P), jnp.float32),
        grid=(2, NS),
        in_specs=[
            pl.BlockSpec((batch, C, P),
                         lambda p, i: (i * (1 - p) + (NS - 1) * p, 0, 0)),
            pl.BlockSpec((9, Cout, C), lambda p, i: (0, 0, 0)),
            pl.BlockSpec((Cout, 1), lambda p, i: (0, 0)),
            pl.BlockSpec((Cout, 1), lambda p, i: (0, 0)),
        ],
        out_specs=pl.BlockSpec((batch, Cout, P), lambda p, i: (i * p, 0, 0)),
        scratch_shapes=[
            pltpu.VMEM((C, _ALIGN + P + _ALIGN), jnp.bfloat16),
            pltpu.VMEM((N, Cout, P), jnp.bfloat16),
            pltpu.VMEM((Cout, 2), jnp.float32),
            pltpu.VMEM((16, P), jnp.float32),
        ],
        compiler_params=cparams,
    )(x_flat, wt, gamma.astype(jnp.float32).reshape(Cout, 1),
      beta.astype(jnp.float32).reshape(Cout, 1))
    return out_flat.reshape(N, Cout, H, W)
